# baseline (speedup 1.0000x reference)
import functools

import jax
import jax.numpy as jnp
from jax.experimental import pallas as pl
from jax.experimental.pallas import tpu as pltpu

_K = 64


def _encode_select_body(x_ref, bpre_ref, W_ref, benc_ref, out_ref,
                        hi16_ref, lo16_ref, tmp16_ref, lo_ref, hi_ref,
                        thi_ref, kp_ref, *, ni, nj, hb):
    i = pl.program_id(0)
    j = pl.program_id(1)

    @pl.when(i < ni)
    def _matmul_block():
        xc = x_ref[...] - bpre_ref[...]
        ht = jax.lax.dot_general(
            W_ref[...], xc, (((1,), (1,)), ((), ())),
            preferred_element_type=jnp.float32)
        ht = ht + benc_ref[...]
        iu = jax.lax.bitcast_convert_type(ht, jnp.int32)
        key = jnp.where(iu < 0, iu ^ jnp.int32(0x7FFFFFFF), iu)
        base = pl.multiple_of(j * hb, hb)
        hi16_ref[i % 2, pl.ds(base, hb), :] = (key >> 16).astype(jnp.int16)
        lo16_ref[i % 2, pl.ds(base, hb), :] = (
            (key & 0xFFFF) ^ 0x8000).astype(jnp.int16)

    nsplit = max(nj // 2, 1)
    it1 = -(-16 // nsplit)
    it2 = -(-16 // max(nj - nsplit, 1))

    @pl.when(i > 0)
    def _select_prev():
        pj = (i - 1) % 2
        hdim = out_ref.shape[1]
        chw = min(1024, hdim)
        nch = hdim // chw
        nsl = chw // 128

        @pl.when(j == 0)
        def _init1():
            lo_ref[...] = jnp.full(lo_ref.shape, jnp.int32(-32768))
            hi_ref[...] = jnp.full(hi_ref.shape, jnp.int32(32767))

        def _count16(chunk, m16):
            acc = None
            for c in range(nch):
                part = chunk(c)
                one = jnp.where(part >= m16, jnp.int16(1), jnp.int16(0))
                acc = one if acc is None else acc + one
            acc2 = None
            for s in range(nsl):
                slab = acc[s * 128:(s + 1) * 128, :]
                acc2 = slab if acc2 is None else acc2 + slab
            return jnp.sum(acc2.astype(jnp.int32), axis=0, keepdims=True)

        def _iters(chunk, n, target):
            for _ in range(n):
                lo = lo_ref[...]
                hi = hi_ref[...]
                mid = (lo >> 1) + (hi >> 1) + ((lo & 1) | (hi & 1))
                cnt = _count16(chunk, mid.astype(jnp.int16))
                take = cnt >= target
                lo_ref[...] = jnp.where(take, mid, lo)
                hi_ref[...] = jnp.where(take, hi, mid - 1)

        def _hi_chunk(c):
            return hi16_ref[pj, c * chw:(c + 1) * chw, :]

        def _tmp_chunk(c):
            return tmp16_ref[c * chw:(c + 1) * chw, :]

        @pl.when(j < nsplit)
        def _phase1():
            _iters(_hi_chunk, it1, _K)

        @pl.when(j == nsplit)
        def _build():
            t16 = lo_ref[...]
            thi_ref[...] = t16
            cgt = _count16(_hi_chunk, (t16 + 1).astype(jnp.int16))
            kp_ref[...] = _K - cgt
            t16s = t16.astype(jnp.int16)
            for c in range(nch):
                sl = slice(c * chw, (c + 1) * chw)
                eq = hi16_ref[pj, sl, :] == t16s
                tmp16_ref[sl, :] = jnp.where(eq, lo16_ref[pj, sl, :],
                                             jnp.int16(-32768))
            lo_ref[...] = jnp.full(lo_ref.shape, jnp.int32(-32768))
            hi_ref[...] = jnp.full(hi_ref.shape, jnp.int32(32767))

        @pl.when(j >= nsplit)
        def _phase2():
            _iters(_tmp_chunk, it2, kp_ref[...])

        @pl.when(j == nj - 1)
        def _write():
            t16 = thi_ref[...].astype(jnp.int16)
            tlow = lo_ref[...].astype(jnp.int16)
            for c in range(nch):
                sl = slice(c * chw, (c + 1) * chw)
                khi = hi16_ref[pj, sl, :]
                klo = lo16_ref[pj, sl, :]
                mask = (khi > t16) | ((khi == t16) & (klo >= tlow))
                khi32 = khi.astype(jnp.int32)
                u16 = (klo.astype(jnp.int32) ^ 0x8000) & 0xFFFF
                key = (khi32 << 16) | u16
                h = jax.lax.bitcast_convert_type(
                    jnp.where(key < 0, key ^ jnp.int32(0x7FFFFFFF), key),
                    jnp.float32)
                res = jnp.where(mask, jnp.maximum(h, 0.0), 0.0)
                out_ref[:, sl] = res.T


def _decode_body(hs_ref, W_ref, bpre_ref, out_ref):
    k = pl.program_id(1)

    @pl.when(k == 0)
    def _init():
        out_ref[...] = jnp.broadcast_to(bpre_ref[...], out_ref.shape)

    out_ref[...] += jax.lax.dot_general(
        hs_ref[...], W_ref[...], (((1,), (1,)), ((), ())),
        preferred_element_type=jnp.float32)


def kernel(x, b_pre, W_enc, b_enc, W_dec):
    bsz, d = x.shape
    hdim = W_enc.shape[0]

    r1 = min(128, bsz)
    hb = min(1024, hdim)
    nj = hdim // hb
    ni = bsz // r1

    h_sparse = pl.pallas_call(
        functools.partial(_encode_select_body, ni=ni, nj=nj, hb=hb),
        grid=(ni + 1, nj),
        in_specs=[
            pl.BlockSpec((r1, d), lambda i, j: (jnp.minimum(i, ni - 1), 0)),
            pl.BlockSpec((1, d), lambda i, j: (0, 0)),
            pl.BlockSpec((hb, d), lambda i, j: (j, 0)),
            pl.BlockSpec((hb, 1), lambda i, j: (j, 0)),
        ],
        out_specs=pl.BlockSpec((r1, hdim),
                               lambda i, j: (jnp.maximum(i - 1, 0), 0)),
        out_shape=jax.ShapeDtypeStruct((bsz, hdim), jnp.float32),
        scratch_shapes=[
            pltpu.VMEM((2, hdim, r1), jnp.int16),
            pltpu.VMEM((2, hdim, r1), jnp.int16),
            pltpu.VMEM((hdim, r1), jnp.int16),
            pltpu.VMEM((1, r1), jnp.int32),
            pltpu.VMEM((1, r1), jnp.int32),
            pltpu.VMEM((1, r1), jnp.int32),
            pltpu.VMEM((1, r1), jnp.int32),
        ],
    )(x, b_pre[None], W_enc, b_enc[:, None])

    r2 = min(1024, bsz)
    kb = min(1024, hdim)
    nk = hdim // kb
    x_rec = pl.pallas_call(
        _decode_body,
        grid=(bsz // r2, nk),
        in_specs=[
            pl.BlockSpec((r2, kb), lambda i, k: (i, k)),
            pl.BlockSpec((d, kb), lambda i, k: (0, k)),
            pl.BlockSpec((1, d), lambda i, k: (0, 0)),
        ],
        out_specs=pl.BlockSpec((r2, d), lambda i, k: (i, 0)),
        out_shape=jax.ShapeDtypeStruct((bsz, d), jnp.float32),
    )(h_sparse, W_dec, b_pre[None])

    return (x_rec, h_sparse)


# speedup vs baseline: 1.1956x; 1.1956x over previous
import functools

import jax
import jax.numpy as jnp
from jax.experimental import pallas as pl
from jax.experimental.pallas import tpu as pltpu

_K = 64


def _encode_select_body(x_ref, W_ref, benc_ref, out_ref,
                        hi16_ref, lo16_ref, tmp16_ref, lo_ref, hi_ref,
                        thi_ref, kp_ref, *, ni, nj, hb):
    i = pl.program_id(0)
    j = pl.program_id(1)

    @pl.when(i < ni)
    def _matmul_block():
        h = jax.lax.dot_general(
            x_ref[...], W_ref[...], (((1,), (1,)), ((), ())),
            preferred_element_type=jnp.float32)
        h = h + benc_ref[...]
        iu = jax.lax.bitcast_convert_type(h, jnp.int32)
        key = jnp.where(iu < 0, iu ^ jnp.int32(0x7FFFFFFF), iu)
        base = pl.multiple_of(j * hb, hb)
        hi16_ref[i % 2, :, pl.ds(base, hb)] = (key >> 16).astype(jnp.int16)
        lo16_ref[i % 2, :, pl.ds(base, hb)] = (
            (key & 0xFFFF) ^ 0x8000).astype(jnp.int16)

    nsplit = max(nj // 2, 1)
    it1 = -(-16 // nsplit)
    it2 = -(-16 // max(nj - nsplit, 1))

    @pl.when(i > 0)
    def _select_prev():
        pj = (i - 1) % 2
        hdim = out_ref.shape[1]
        chw = min(1024, hdim)
        nch = hdim // chw
        nsl = chw // 128

        @pl.when(j == 0)
        def _init1():
            lo_ref[...] = jnp.full(lo_ref.shape, jnp.int32(-32768))
            hi_ref[...] = jnp.full(hi_ref.shape, jnp.int32(32767))

        def _count16(chunk, m16):
            acc = None
            for c in range(nch):
                part = chunk(c)
                one = jnp.where(part >= m16, jnp.int16(1), jnp.int16(0))
                acc = one if acc is None else acc + one
            acc2 = None
            for s in range(nsl):
                slab = acc[:, s * 128:(s + 1) * 128]
                acc2 = slab if acc2 is None else acc2 + slab
            return jnp.sum(acc2.astype(jnp.int32), axis=1, keepdims=True)

        def _iters(chunk, n, target):
            for _ in range(n):
                lo = lo_ref[...]
                hi = hi_ref[...]
                mid = (lo >> 1) + (hi >> 1) + ((lo & 1) | (hi & 1))
                cnt = _count16(chunk, mid.astype(jnp.int16))
                take = cnt >= target
                lo_ref[...] = jnp.where(take, mid, lo)
                hi_ref[...] = jnp.where(take, hi, mid - 1)

        def _hi_chunk(c):
            return hi16_ref[pj, :, c * chw:(c + 1) * chw]

        def _tmp_chunk(c):
            return tmp16_ref[:, c * chw:(c + 1) * chw]

        @pl.when(j < nsplit)
        def _phase1():
            _iters(_hi_chunk, it1, _K)

        @pl.when(j == nsplit)
        def _build():
            t16 = lo_ref[...]
            thi_ref[...] = t16
            cgt = _count16(_hi_chunk, (t16 + 1).astype(jnp.int16))
            kp_ref[...] = _K - cgt
            t16s = t16.astype(jnp.int16)
            for c in range(nch):
                sl = slice(c * chw, (c + 1) * chw)
                eq = hi16_ref[pj, :, sl] == t16s
                tmp16_ref[:, sl] = jnp.where(eq, lo16_ref[pj, :, sl],
                                             jnp.int16(-32768))
            lo_ref[...] = jnp.full(lo_ref.shape, jnp.int32(-32768))
            hi_ref[...] = jnp.full(hi_ref.shape, jnp.int32(32767))

        @pl.when(j >= nsplit)
        def _phase2():
            _iters(_tmp_chunk, it2, kp_ref[...])

        @pl.when(j == nj - 1)
        def _write():
            t16 = thi_ref[...].astype(jnp.int16)
            tlow = lo_ref[...].astype(jnp.int16)
            for c in range(nch):
                sl = slice(c * chw, (c + 1) * chw)
                khi = hi16_ref[pj, :, sl]
                klo = lo16_ref[pj, :, sl]
                mask = (khi > t16) | ((khi == t16) & (klo >= tlow))
                khi32 = khi.astype(jnp.int32)
                u16 = (klo.astype(jnp.int32) ^ 0x8000) & 0xFFFF
                key = (khi32 << 16) | u16
                h = jax.lax.bitcast_convert_type(
                    jnp.where(key < 0, key ^ jnp.int32(0x7FFFFFFF), key),
                    jnp.float32)
                out_ref[:, sl] = jnp.where(mask, jnp.maximum(h, 0.0), 0.0)


def _decode_body(hs_ref, W_ref, bpre_ref, out_ref):
    k = pl.program_id(1)

    @pl.when(k == 0)
    def _init():
        out_ref[...] = jnp.broadcast_to(bpre_ref[...], out_ref.shape)

    out_ref[...] += jax.lax.dot_general(
        hs_ref[...].astype(jnp.bfloat16), W_ref[...], (((1,), (1,)), ((), ())),
        preferred_element_type=jnp.float32)


def kernel(x, b_pre, W_enc, b_enc, W_dec):
    bsz, d = x.shape
    hdim = W_enc.shape[0]

    r1 = min(128, bsz)
    hb = min(1024, hdim)
    nj = hdim // hb
    ni = bsz // r1

    xb = (x - b_pre).astype(jnp.bfloat16)
    Wb = W_enc.astype(jnp.bfloat16)
    h_sparse = pl.pallas_call(
        functools.partial(_encode_select_body, ni=ni, nj=nj, hb=hb),
        grid=(ni + 1, nj),
        in_specs=[
            pl.BlockSpec((r1, d), lambda i, j: (jnp.minimum(i, ni - 1), 0)),
            pl.BlockSpec((hb, d), lambda i, j: (j, 0)),
            pl.BlockSpec((1, hb), lambda i, j: (0, j)),
        ],
        out_specs=pl.BlockSpec((r1, hdim),
                               lambda i, j: (jnp.maximum(i - 1, 0), 0)),
        out_shape=jax.ShapeDtypeStruct((bsz, hdim), jnp.float32),
        scratch_shapes=[
            pltpu.VMEM((2, r1, hdim), jnp.int16),
            pltpu.VMEM((2, r1, hdim), jnp.int16),
            pltpu.VMEM((r1, hdim), jnp.int16),
            pltpu.VMEM((r1, 1), jnp.int32),
            pltpu.VMEM((r1, 1), jnp.int32),
            pltpu.VMEM((r1, 1), jnp.int32),
            pltpu.VMEM((r1, 1), jnp.int32),
        ],
    )(xb, Wb, b_enc[None])

    r2 = min(1024, bsz)
    kb = min(1024, hdim)
    nk = hdim // kb
    x_rec = pl.pallas_call(
        _decode_body,
        grid=(bsz // r2, nk),
        in_specs=[
            pl.BlockSpec((r2, kb), lambda i, k: (i, k)),
            pl.BlockSpec((d, kb), lambda i, k: (0, k)),
            pl.BlockSpec((1, d), lambda i, k: (0, 0)),
        ],
        out_specs=pl.BlockSpec((r2, d), lambda i, k: (i, 0)),
        out_shape=jax.ShapeDtypeStruct((bsz, d), jnp.float32),
    )(h_sparse, W_dec.astype(jnp.bfloat16), b_pre[None])

    return (x_rec, h_sparse)


# speedup vs baseline: 1.2615x; 1.0551x over previous
import functools

import jax
import jax.numpy as jnp
from jax.experimental import pallas as pl
from jax.experimental.pallas import tpu as pltpu

_K = 64


def _encode_select_body(x_ref, W_ref, benc_ref, out_ref,
                        hi16_ref, lo16_ref, tmp16_ref, lo_ref, hi_ref,
                        thi_ref, kp_ref, *, ni, nj, hb):
    i = pl.program_id(0)
    j = pl.program_id(1)

    @pl.when(i < ni)
    def _matmul_block():
        h = jax.lax.dot_general(
            x_ref[...], W_ref[...], (((1,), (1,)), ((), ())),
            preferred_element_type=jnp.float32)
        h = h + benc_ref[...]
        iu = jax.lax.bitcast_convert_type(h, jnp.int32)
        key = jnp.where(iu < 0, iu ^ jnp.int32(0x7FFFFFFF), iu)
        base = pl.multiple_of(j * hb, hb)
        hi16_ref[i % 2, :, pl.ds(base, hb)] = (key >> 16).astype(jnp.int16)
        lo16_ref[i % 2, :, pl.ds(base, hb)] = (
            (key & 0xFFFF) ^ 0x8000).astype(jnp.int16)

    nsplit = max(nj // 2, 1)
    it1 = -(-16 // nsplit)
    it2 = -(-16 // max(nj - nsplit, 1))

    @pl.when(i > 0)
    def _select_prev():
        pj = (i - 1) % 2
        hdim = out_ref.shape[1]
        chw = min(1024, hdim)
        nch = hdim // chw
        nsl = chw // 128

        @pl.when(j == 0)
        def _init1():
            lo_ref[...] = jnp.full(lo_ref.shape, jnp.int32(-32768))
            hi_ref[...] = jnp.full(hi_ref.shape, jnp.int32(32767))

        def _count16(chunk, m16):
            acc = None
            for c in range(nch):
                part = chunk(c)
                one = jnp.where(part >= m16, jnp.int16(1), jnp.int16(0))
                acc = one if acc is None else acc + one
            acc2 = None
            for s in range(nsl):
                slab = acc[:, s * 128:(s + 1) * 128]
                acc2 = slab if acc2 is None else acc2 + slab
            return jnp.sum(acc2.astype(jnp.int32), axis=1, keepdims=True)

        def _iters(chunk, n, target):
            for _ in range(n):
                lo = lo_ref[...]
                hi = hi_ref[...]
                mid = (lo >> 1) + (hi >> 1) + ((lo & 1) | (hi & 1))
                cnt = _count16(chunk, mid.astype(jnp.int16))
                take = cnt >= target
                lo_ref[...] = jnp.where(take, mid, lo)
                hi_ref[...] = jnp.where(take, hi, mid - 1)

        def _hi_chunk(c):
            return hi16_ref[pj, :, c * chw:(c + 1) * chw]

        def _tmp_chunk(c):
            return tmp16_ref[:, c * chw:(c + 1) * chw]

        @pl.when(j < nsplit)
        def _phase1():
            _iters(_hi_chunk, it1, _K)

        @pl.when(j == nsplit)
        def _build():
            t16 = lo_ref[...]
            thi_ref[...] = t16
            cgt = _count16(_hi_chunk, (t16 + 1).astype(jnp.int16))
            kp_ref[...] = _K - cgt
            t16s = t16.astype(jnp.int16)
            for c in range(nch):
                sl = slice(c * chw, (c + 1) * chw)
                eq = hi16_ref[pj, :, sl] == t16s
                tmp16_ref[:, sl] = jnp.where(eq, lo16_ref[pj, :, sl],
                                             jnp.int16(-32768))
            lo_ref[...] = jnp.full(lo_ref.shape, jnp.int32(-32768))
            hi_ref[...] = jnp.full(hi_ref.shape, jnp.int32(32767))

        @pl.when(j >= nsplit)
        def _phase2():
            _iters(_tmp_chunk, it2, kp_ref[...])

        @pl.when(j == nj - 1)
        def _write():
            t16 = thi_ref[...].astype(jnp.int16)
            tlow = lo_ref[...].astype(jnp.int16)
            for c in range(nch):
                sl = slice(c * chw, (c + 1) * chw)
                khi = hi16_ref[pj, :, sl]
                klo = lo16_ref[pj, :, sl]
                mask = (khi > t16) | ((khi == t16) & (klo >= tlow))
                khi32 = khi.astype(jnp.int32)
                u16 = (klo.astype(jnp.int32) ^ 0x8000) & 0xFFFF
                key = (khi32 << 16) | u16
                h = jax.lax.bitcast_convert_type(
                    jnp.where(key < 0, key ^ jnp.int32(0x7FFFFFFF), key),
                    jnp.float32)
                out_ref[:, sl] = jnp.where(mask, jnp.maximum(h, 0.0), 0.0)


def _decode_body(hs_ref, W_ref, bpre_ref, out_ref):
    k = pl.program_id(1)

    @pl.when(k == 0)
    def _init():
        out_ref[...] = jnp.broadcast_to(bpre_ref[...], out_ref.shape)

    out_ref[...] += jax.lax.dot_general(
        hs_ref[...].astype(jnp.bfloat16), W_ref[...], (((1,), (1,)), ((), ())),
        preferred_element_type=jnp.float32)


def kernel(x, b_pre, W_enc, b_enc, W_dec):
    bsz, d = x.shape
    hdim = W_enc.shape[0]

    r1 = min(128, bsz)
    hb = min(2048, hdim)
    nj = hdim // hb
    ni = bsz // r1

    xb = (x - b_pre).astype(jnp.bfloat16)
    Wb = W_enc.astype(jnp.bfloat16)
    h_sparse = pl.pallas_call(
        functools.partial(_encode_select_body, ni=ni, nj=nj, hb=hb),
        grid=(ni + 1, nj),
        in_specs=[
            pl.BlockSpec((r1, d), lambda i, j: (jnp.minimum(i, ni - 1), 0)),
            pl.BlockSpec((hb, d), lambda i, j: (j, 0)),
            pl.BlockSpec((1, hb), lambda i, j: (0, j)),
        ],
        out_specs=pl.BlockSpec((r1, hdim),
                               lambda i, j: (jnp.maximum(i - 1, 0), 0)),
        out_shape=jax.ShapeDtypeStruct((bsz, hdim), jnp.float32),
        scratch_shapes=[
            pltpu.VMEM((2, r1, hdim), jnp.int16),
            pltpu.VMEM((2, r1, hdim), jnp.int16),
            pltpu.VMEM((r1, hdim), jnp.int16),
            pltpu.VMEM((r1, 1), jnp.int32),
            pltpu.VMEM((r1, 1), jnp.int32),
            pltpu.VMEM((r1, 1), jnp.int32),
            pltpu.VMEM((r1, 1), jnp.int32),
        ],
    )(xb, Wb, b_enc[None])

    r2 = min(1024, bsz)
    kb = min(1024, hdim)
    nk = hdim // kb
    x_rec = pl.pallas_call(
        _decode_body,
        grid=(bsz // r2, nk),
        in_specs=[
            pl.BlockSpec((r2, kb), lambda i, k: (i, k)),
            pl.BlockSpec((d, kb), lambda i, k: (0, k)),
            pl.BlockSpec((1, d), lambda i, k: (0, 0)),
        ],
        out_specs=pl.BlockSpec((r2, d), lambda i, k: (i, 0)),
        out_shape=jax.ShapeDtypeStruct((bsz, d), jnp.float32),
    )(h_sparse, W_dec.astype(jnp.bfloat16), b_pre[None])

    return (x_rec, h_sparse)


# speedup vs baseline: 1.7728x; 1.4054x over previous
import functools

import jax
import jax.numpy as jnp
from jax.experimental import pallas as pl
from jax.experimental.pallas import tpu as pltpu

_K = 64


def _encode_mm_body(x_ref, W_ref, benc_ref, hi_ref, lo_ref):
    h = jax.lax.dot_general(
        x_ref[...], W_ref[...], (((1,), (1,)), ((), ())),
        preferred_element_type=jnp.float32)
    h = h + benc_ref[...]
    iu = jax.lax.bitcast_convert_type(h, jnp.int32)
    key = jnp.where(iu < 0, iu ^ jnp.int32(0x7FFFFFFF), iu)
    hi_ref[...] = (key >> 16).astype(jnp.int16)
    lo_ref[...] = ((key & 0xFFFF) ^ 0x8000).astype(jnp.int16)


def _select_body(hi_ref, lo_ref, out_ref, tmp_ref):
    hdim = out_ref.shape[1]
    chw = min(1024, hdim)
    nch = hdim // chw
    nsl = chw // 128

    def _count16(chunk, m16):
        acc = None
        for c in range(nch):
            part = chunk(c)
            one = jnp.where(part >= m16, jnp.int16(1), jnp.int16(0))
            acc = one if acc is None else acc + one
        acc2 = None
        for s in range(nsl):
            slab = acc[:, s * 128:(s + 1) * 128]
            acc2 = slab if acc2 is None else acc2 + slab
        return jnp.sum(acc2.astype(jnp.int32), axis=1, keepdims=True)

    def _search(chunk, target):
        rows = out_ref.shape[0]
        lo = jnp.full((rows, 1), jnp.int32(-32768))
        hi = jnp.full((rows, 1), jnp.int32(32767))
        for _ in range(16):
            mid = (lo >> 1) + (hi >> 1) + ((lo & 1) | (hi & 1))
            cnt = _count16(chunk, mid.astype(jnp.int16))
            take = cnt >= target
            lo = jnp.where(take, mid, lo)
            hi = jnp.where(take, hi, mid - 1)
        return lo

    def _hi_chunk(c):
        return hi_ref[:, c * chw:(c + 1) * chw]

    def _tmp_chunk(c):
        return tmp_ref[:, c * chw:(c + 1) * chw]

    t16 = _search(_hi_chunk, _K)
    cgt = _count16(_hi_chunk, (t16 + 1).astype(jnp.int16))
    kprime = _K - cgt
    t16s = t16.astype(jnp.int16)
    for c in range(nch):
        sl = slice(c * chw, (c + 1) * chw)
        eq = hi_ref[:, sl] == t16s
        tmp_ref[:, sl] = jnp.where(eq, lo_ref[:, sl], jnp.int16(-32768))
    tlow = _search(_tmp_chunk, kprime).astype(jnp.int16)

    for c in range(nch):
        sl = slice(c * chw, (c + 1) * chw)
        khi = hi_ref[:, sl]
        klo = lo_ref[:, sl]
        mask = (khi > t16s) | ((khi == t16s) & (klo >= tlow))
        khi32 = khi.astype(jnp.int32)
        u16 = (klo.astype(jnp.int32) ^ 0x8000) & 0xFFFF
        key = (khi32 << 16) | u16
        h = jax.lax.bitcast_convert_type(
            jnp.where(key < 0, key ^ jnp.int32(0x7FFFFFFF), key),
            jnp.float32)
        out_ref[:, sl] = jnp.where(mask, jnp.maximum(h, 0.0), 0.0)


def _decode_body(hs_ref, W_ref, bpre_ref, out_ref):
    k = pl.program_id(1)

    @pl.when(k == 0)
    def _init():
        out_ref[...] = jnp.broadcast_to(bpre_ref[...], out_ref.shape)

    out_ref[...] += jax.lax.dot_general(
        hs_ref[...].astype(jnp.bfloat16), W_ref[...],
        (((1,), (1,)), ((), ())),
        preferred_element_type=jnp.float32)


def kernel(x, b_pre, W_enc, b_enc, W_dec):
    bsz, d = x.shape
    hdim = W_enc.shape[0]

    xb = (x - b_pre).astype(jnp.bfloat16)
    Wb = W_enc.astype(jnp.bfloat16)

    r1 = min(1024, bsz)
    hb = min(2048, hdim)
    hi_p, lo_p = pl.pallas_call(
        _encode_mm_body,
        grid=(bsz // r1, hdim // hb),
        in_specs=[
            pl.BlockSpec((r1, d), lambda i, j: (i, 0)),
            pl.BlockSpec((hb, d), lambda i, j: (j, 0)),
            pl.BlockSpec((1, hb), lambda i, j: (0, j)),
        ],
        out_specs=[
            pl.BlockSpec((r1, hb), lambda i, j: (i, j)),
            pl.BlockSpec((r1, hb), lambda i, j: (i, j)),
        ],
        out_shape=[
            jax.ShapeDtypeStruct((bsz, hdim), jnp.int16),
            jax.ShapeDtypeStruct((bsz, hdim), jnp.int16),
        ],
    )(xb, Wb, b_enc[None])

    r2 = min(128, bsz)
    h_sparse = pl.pallas_call(
        _select_body,
        grid=(bsz // r2,),
        in_specs=[
            pl.BlockSpec((r2, hdim), lambda i: (i, 0)),
            pl.BlockSpec((r2, hdim), lambda i: (i, 0)),
        ],
        out_specs=pl.BlockSpec((r2, hdim), lambda i: (i, 0)),
        out_shape=jax.ShapeDtypeStruct((bsz, hdim), jnp.float32),
        scratch_shapes=[pltpu.VMEM((r2, hdim), jnp.int16)],
    )(hi_p, lo_p)

    r3 = min(1024, bsz)
    kb = min(1024, hdim)
    x_rec = pl.pallas_call(
        _decode_body,
        grid=(bsz // r3, hdim // kb),
        in_specs=[
            pl.BlockSpec((r3, kb), lambda i, k: (i, k)),
            pl.BlockSpec((d, kb), lambda i, k: (0, k)),
            pl.BlockSpec((1, d), lambda i, k: (0, 0)),
        ],
        out_specs=pl.BlockSpec((r3, d), lambda i, k: (i, 0)),
        out_shape=jax.ShapeDtypeStruct((bsz, d), jnp.float32),
    )(h_sparse, W_dec.astype(jnp.bfloat16), b_pre[None])

    return (x_rec, h_sparse)


# speedup vs baseline: 1.7741x; 1.0007x over previous
import functools

import jax
import jax.numpy as jnp
from jax.experimental import pallas as pl
from jax.experimental.pallas import tpu as pltpu

_K = 64


def _encode_mm_body(x_ref, W_ref, benc_ref, hi_ref, lo_ref):
    h = jax.lax.dot_general(
        x_ref[...], W_ref[...], (((1,), (1,)), ((), ())),
        preferred_element_type=jnp.float32)
    h = h + benc_ref[...]
    iu = jax.lax.bitcast_convert_type(h, jnp.int32)
    key = jnp.where(iu < 0, iu ^ jnp.int32(0x7FFFFFFF), iu)
    hi_ref[...] = (key >> 16).astype(jnp.int16)
    lo_ref[...] = ((key & 0xFFFF) ^ 0x8000).astype(jnp.int16)


def _select_body(hi_ref, lo_ref, out_ref, tmp_ref):
    hdim = out_ref.shape[1]
    chw = min(1024, hdim)
    nch = hdim // chw
    nsl = chw // 128

    def _count16(chunk, m16):
        acc = None
        for c in range(nch):
            part = chunk(c)
            one = (part >= m16).astype(jnp.int16)
            acc = one if acc is None else acc + one
        acc2 = None
        for s in range(nsl):
            slab = acc[:, s * 128:(s + 1) * 128]
            acc2 = slab if acc2 is None else acc2 + slab
        return jnp.sum(acc2.astype(jnp.int32), axis=1, keepdims=True)

    def _search(chunk, target):
        rows = out_ref.shape[0]
        lo = jnp.full((rows, 1), jnp.int32(-32768))
        hi = jnp.full((rows, 1), jnp.int32(32767))
        for _ in range(16):
            mid = (lo >> 1) + (hi >> 1) + ((lo & 1) | (hi & 1))
            cnt = _count16(chunk, mid.astype(jnp.int16))
            take = cnt >= target
            lo = jnp.where(take, mid, lo)
            hi = jnp.where(take, hi, mid - 1)
        return lo

    def _hi_chunk(c):
        return hi_ref[:, c * chw:(c + 1) * chw]

    def _tmp_chunk(c):
        return tmp_ref[:, c * chw:(c + 1) * chw]

    t16 = _search(_hi_chunk, _K)
    cgt = _count16(_hi_chunk, (t16 + 1).astype(jnp.int16))
    kprime = _K - cgt
    t16s = t16.astype(jnp.int16)
    for c in range(nch):
        sl = slice(c * chw, (c + 1) * chw)
        eq = hi_ref[:, sl] == t16s
        tmp_ref[:, sl] = jnp.where(eq, lo_ref[:, sl], jnp.int16(-32768))
    tlow = _search(_tmp_chunk, kprime).astype(jnp.int16)

    for c in range(nch):
        sl = slice(c * chw, (c + 1) * chw)
        khi = hi_ref[:, sl]
        klo = lo_ref[:, sl]
        mask = (khi > t16s) | ((khi == t16s) & (klo >= tlow))
        khi32 = khi.astype(jnp.int32)
        u16 = (klo.astype(jnp.int32) ^ 0x8000) & 0xFFFF
        key = (khi32 << 16) | u16
        h = jax.lax.bitcast_convert_type(
            jnp.where(key < 0, key ^ jnp.int32(0x7FFFFFFF), key),
            jnp.float32)
        out_ref[:, sl] = jnp.where(mask, jnp.maximum(h, 0.0), 0.0)


def _decode_body(hs_ref, W_ref, bpre_ref, out_ref):
    k = pl.program_id(1)

    @pl.when(k == 0)
    def _init():
        out_ref[...] = jnp.broadcast_to(bpre_ref[...], out_ref.shape)

    out_ref[...] += jax.lax.dot_general(
        hs_ref[...].astype(jnp.bfloat16), W_ref[...],
        (((1,), (1,)), ((), ())),
        preferred_element_type=jnp.float32)


def kernel(x, b_pre, W_enc, b_enc, W_dec):
    bsz, d = x.shape
    hdim = W_enc.shape[0]

    xb = (x - b_pre).astype(jnp.bfloat16)
    Wb = W_enc.astype(jnp.bfloat16)

    r1 = min(1024, bsz)
    hb = min(2048, hdim)
    hi_p, lo_p = pl.pallas_call(
        _encode_mm_body,
        grid=(bsz // r1, hdim // hb),
        in_specs=[
            pl.BlockSpec((r1, d), lambda i, j: (i, 0)),
            pl.BlockSpec((hb, d), lambda i, j: (j, 0)),
            pl.BlockSpec((1, hb), lambda i, j: (0, j)),
        ],
        out_specs=[
            pl.BlockSpec((r1, hb), lambda i, j: (i, j)),
            pl.BlockSpec((r1, hb), lambda i, j: (i, j)),
        ],
        out_shape=[
            jax.ShapeDtypeStruct((bsz, hdim), jnp.int16),
            jax.ShapeDtypeStruct((bsz, hdim), jnp.int16),
        ],
    )(xb, Wb, b_enc[None])

    r2 = min(128, bsz)
    h_sparse = pl.pallas_call(
        _select_body,
        grid=(bsz // r2,),
        in_specs=[
            pl.BlockSpec((r2, hdim), lambda i: (i, 0)),
            pl.BlockSpec((r2, hdim), lambda i: (i, 0)),
        ],
        out_specs=pl.BlockSpec((r2, hdim), lambda i: (i, 0)),
        out_shape=jax.ShapeDtypeStruct((bsz, hdim), jnp.float32),
        scratch_shapes=[pltpu.VMEM((r2, hdim), jnp.int16)],
    )(hi_p, lo_p)

    r3 = min(1024, bsz)
    kb = min(1024, hdim)
    x_rec = pl.pallas_call(
        _decode_body,
        grid=(bsz // r3, hdim // kb),
        in_specs=[
            pl.BlockSpec((r3, kb), lambda i, k: (i, k)),
            pl.BlockSpec((d, kb), lambda i, k: (0, k)),
            pl.BlockSpec((1, d), lambda i, k: (0, 0)),
        ],
        out_specs=pl.BlockSpec((r3, d), lambda i, k: (i, 0)),
        out_shape=jax.ShapeDtypeStruct((bsz, d), jnp.float32),
    )(h_sparse, W_dec.astype(jnp.bfloat16), b_pre[None])

    return (x_rec, h_sparse)
